# baseline (speedup 1.0000x reference)
import functools

import jax
import jax.numpy as jnp
from jax.experimental import pallas as pl
from jax.experimental.pallas import tpu as pltpu

_THRESHOLD = 0.5
_NEG_POS_RATIO = 3.0
_ALPHA = 1.0
_SUB = 8
_CH = 384


def _stage1_body(locs_ref, scores_ref, boxes_ref, labels_ref, priors_ref,
                 conf_ref, stats_ref, ov_s, ofp_s, obj_s, *, n_obj, p_real):
    s, l = priors_ref.shape[1], priors_ref.shape[2]
    p2 = s * l
    nch = l // _CH

    def box_scalars(j):
        bx0 = boxes_ref[0, j, 0]
        by0 = boxes_ref[0, j, 1]
        bx1 = boxes_ref[0, j, 2]
        by1 = boxes_ref[0, j, 3]
        return bx0, by0, bx1, by1

    for c in range(nch):
        cs = slice(c * _CH, (c + 1) * _CH)
        pcx = priors_ref[0, :, cs]
        pcy = priors_ref[1, :, cs]
        pw = priors_ref[2, :, cs]
        ph = priors_ref[3, :, cs]
        px0 = pcx - pw / 2.0
        py0 = pcy - ph / 2.0
        px1 = pcx + pw / 2.0
        py1 = pcy + ph / 2.0
        a2 = (px1 - px0) * (py1 - py0)
        vals = []
        idxs = []
        for j in range(n_obj):
            bx0, by0, bx1, by1 = box_scalars(j)
            iw = jnp.maximum(
                jnp.minimum(bx1, px1) - jnp.maximum(bx0, px0), 0.0)
            ih = jnp.maximum(
                jnp.minimum(by1, py1) - jnp.maximum(by0, py0), 0.0)
            inter = iw * ih
            a1 = (bx1 - bx0) * (by1 - by0)
            ov = inter / (a1 + a2 - inter)
            ov_s[j, :, cs] = ov
            vals.append(ov)
            idxs.append(j)
        while len(vals) > 1:
            nv, ni = [], []
            for t in range(0, len(vals), 2):
                keep = vals[t] >= vals[t + 1]
                nv.append(jnp.maximum(vals[t], vals[t + 1]))
                a, b = idxs[t], idxs[t + 1]
                if isinstance(a, int):
                    a = jnp.full((s, _CH), a, jnp.int32)
                ni.append(jnp.where(keep, a, b))
            vals, idxs = nv, ni
        ofp_s[:, cs] = vals[0]
        obj_s[:, cs] = idxs[0]

    p_idx = (jax.lax.broadcasted_iota(jnp.int32, (s, l), 0) * l +
             jax.lax.broadcasted_iota(jnp.int32, (s, l), 1))
    pfo = []
    for j in range(n_obj):
        ovj = ov_s[j]
        rmax = jnp.max(ovj)
        pfo.append(jnp.min(jnp.where(ovj >= rmax, p_idx, p2)))

    acc_npos = jnp.zeros((s, _CH), jnp.float32)
    acc_cpos = jnp.zeros((s, _CH), jnp.float32)
    acc_labs = jnp.zeros((s, _CH), jnp.float32)
    for c in range(nch):
        cs = slice(c * _CH, (c + 1) * _CH)
        p_idx_c = (jax.lax.broadcasted_iota(jnp.int32, (s, _CH), 0) * l +
                   jax.lax.broadcasted_iota(jnp.int32, (s, _CH), 1) + c * _CH)
        ofp_c = ofp_s[:, cs]
        obj_c = obj_s[:, cs]
        fj = [jnp.where(p_idx_c == pfo[j], j, -1) for j in range(n_obj)]
        while len(fj) > 1:
            fj = [jnp.maximum(fj[t], fj[t + 1]) for t in range(0, len(fj), 2)]
        fj = fj[0]
        forced = fj >= 0
        obj_c = jnp.where(forced, fj, obj_c)
        ofp_c = jnp.where(forced, 1.0, ofp_c)
        bbits = [((obj_c >> b) & 1) == 1 for b in range(4)]

        def _mux(table):
            lvl = list(table)
            for b in range(4):
                lvl = [jnp.where(bbits[b], lvl[t + 1], lvl[t])
                       for t in range(0, len(lvl), 2)]
            return lvl[0]

        lab = _mux([labels_ref[0, 0, j] for j in range(n_obj)])
        mx0 = _mux([boxes_ref[0, j, 0] for j in range(n_obj)])
        my0 = _mux([boxes_ref[0, j, 1] for j in range(n_obj)])
        mx1 = _mux([boxes_ref[0, j, 2] for j in range(n_obj)])
        my1 = _mux([boxes_ref[0, j, 3] for j in range(n_obj)])

        tc = jnp.where(ofp_c < _THRESHOLD, 0, lab)
        posm = (tc > 0) & (p_idx_c < p_real)
        posf = jnp.where(posm, 1.0, 0.0)
        acc_npos = acc_npos + posf

        pcx = priors_ref[0, :, cs]
        pcy = priors_ref[1, :, cs]
        pw = priors_ref[2, :, cs]
        ph = priors_ref[3, :, cs]
        cx = (mx0 + mx1) / 2.0
        cy = (my0 + my1) / 2.0
        w = mx1 - mx0
        h = my1 - my0
        g0 = (cx - pcx) / (pw / 10.0)
        g1 = (cy - pcy) / (ph / 10.0)
        g2 = jnp.log(w / pw) * 5.0
        g3 = jnp.log(h / ph) * 5.0
        loc_abs = (jnp.abs(locs_ref[0, 0, :, cs] - g0) +
                   jnp.abs(locs_ref[0, 1, :, cs] - g1) +
                   jnp.abs(locs_ref[0, 2, :, cs] - g2) +
                   jnp.abs(locs_ref[0, 3, :, cs] - g3))
        acc_labs = acc_labs + jnp.where(posm, loc_abs, 0.0)

        l0 = scores_ref[0, 0, :, cs]
        l1 = scores_ref[0, 1, :, cs]
        l2 = scores_ref[0, 2, :, cs]

        def _sp(x):
            return jnp.maximum(x, 0.0) + jnp.log1p(jnp.exp(-jnp.abs(x)))

        s_all = _sp(l0) + _sp(l1) + _sp(l2)
        d = jnp.where(tc == 0, l0,
                      jnp.where(tc == 1, l1,
                                jnp.where(tc == 2, l2, l1 + l2)))
        conf_all = s_all - d
        acc_cpos = acc_cpos + jnp.where(posm, conf_all, 0.0)
        conf_ref[0, :, cs] = jnp.where(
            posm | (p_idx_c >= p_real), 0.0, conf_all)

    n_pos = jnp.sum(acc_npos)
    conf_pos_row = jnp.sum(acc_cpos)
    loc_abs_row = jnp.sum(acc_labs)
    lane = jax.lax.broadcasted_iota(jnp.int32, (1, 128), 1)
    stats_ref[0] = jnp.where(
        lane == 0, n_pos,
        jnp.where(lane == 1, conf_pos_row,
                  jnp.where(lane == 2, loc_abs_row, 0.0)))


def _stage2_body(conf_ref, stats_ref, out_ref, *, p_real):
    b = conf_ref.shape[0]
    conf = conf_ref[...]
    stats = stats_ref[...]
    n_pos = stats[:, 0:1]
    npt = jnp.sum(n_pos)
    kf = jnp.minimum(n_pos * _NEG_POS_RATIO, float(p_real))

    bits = jax.lax.bitcast_convert_type(conf, jnp.int32)
    lo0 = jnp.full((b, 1), -1, jnp.int32)
    hi0 = jnp.full((b, 1), jnp.iinfo(jnp.int32).max - 1, jnp.int32)

    def _cnt(t):
        return jnp.sum(jnp.where(bits > t, 1.0, 0.0), axis=1, keepdims=True)

    def _it4(_, carry):
        lo, hi = carry
        q = (hi - lo) >> 2
        t1 = lo + q
        t2 = lo + 2 * q
        t3 = lo + 3 * q
        a1 = _cnt(t1) >= kf
        a2 = _cnt(t2) >= kf
        a3 = _cnt(t3) >= kf
        nlo = jnp.where(a3, t3, jnp.where(a2, t2, jnp.where(a1, t1, lo)))
        nhi = jnp.where(a1, jnp.where(a2, jnp.where(a3, hi, t3), t2), t1)
        return nlo, nhi

    def _it(_, carry):
        lo, hi = carry
        mid = lo + ((hi - lo) >> 1)
        take_hi = _cnt(mid) >= kf
        return jnp.where(take_hi, mid, lo), jnp.where(take_hi, hi, mid)

    carry = jax.lax.fori_loop(0, 15, _it4, (lo0, hi0))
    _, hi = jax.lax.fori_loop(0, 3, _it, carry)
    vk_bits = hi
    vk = jax.lax.bitcast_convert_type(vk_bits, jnp.float32)
    gt = bits > vk_bits
    cnt_g = jnp.sum(jnp.where(gt, 1.0, 0.0), axis=1, keepdims=True)
    sum_g = jnp.sum(jnp.where(gt, conf, 0.0), axis=1, keepdims=True)
    hard = jnp.where(kf > 0.0, sum_g + vk * (kf - cnt_g), 0.0)

    conf_pos = jnp.sum(stats[:, 1:2])
    loc_abs = jnp.sum(stats[:, 2:3])
    conf_loss = (jnp.sum(hard) + conf_pos) / (1e-10 + npt)
    loc_loss = jnp.where(npt > 0.0,
                         loc_abs / (4.0 * jnp.maximum(npt, 1.0)), 0.0)
    total = conf_loss + _ALPHA * loc_loss

    lane = jax.lax.broadcasted_iota(jnp.int32, (1, 128), 1)
    out_ref[...] = jnp.where(
        lane == 0, total,
        jnp.where(lane == 1, conf_loss,
                  jnp.where(lane == 2, loc_loss, 0.0)))


def kernel(predicted_locs, predicted_scores, boxes, labels, priors_cxcy):
    bsz, p, _ = predicted_scores.shape
    n_obj = boxes.shape[1]
    p2 = p + (-p % (_SUB * 128))
    lsz = p2 // _SUB

    def _fold(x_t):
        pads = [(0, 0)] * (x_t.ndim - 1) + [(0, p2 - p)]
        return jnp.pad(x_t, pads).reshape(x_t.shape[:-1] + (_SUB, lsz))

    locs_f = _fold(jnp.transpose(predicted_locs, (0, 2, 1)))
    scores_f = _fold(jnp.transpose(predicted_scores, (0, 2, 1)))
    priors_f = _fold(jnp.transpose(priors_cxcy, (1, 0)))

    conf_neg, stats = pl.pallas_call(
        functools.partial(_stage1_body, n_obj=n_obj, p_real=p),
        grid=(bsz,),
        in_specs=[
            pl.BlockSpec((1, 4, _SUB, lsz), lambda i: (i, 0, 0, 0)),
            pl.BlockSpec((1, 3, _SUB, lsz), lambda i: (i, 0, 0, 0)),
            pl.BlockSpec((1, n_obj, 4), lambda i: (i, 0, 0),
                         memory_space=pltpu.SMEM),
            pl.BlockSpec((1, 1, n_obj), lambda i: (i, 0, 0),
                         memory_space=pltpu.SMEM),
            pl.BlockSpec((4, _SUB, lsz), lambda i: (0, 0, 0)),
        ],
        out_specs=[
            pl.BlockSpec((1, _SUB, lsz), lambda i: (i, 0, 0)),
            pl.BlockSpec((1, 1, 128), lambda i: (i, 0, 0)),
        ],
        out_shape=[
            jax.ShapeDtypeStruct((bsz, _SUB, lsz), jnp.float32),
            jax.ShapeDtypeStruct((bsz, 1, 128), jnp.float32),
        ],
        scratch_shapes=[
            pltpu.VMEM((n_obj, _SUB, lsz), jnp.float32),
            pltpu.VMEM((_SUB, lsz), jnp.float32),
            pltpu.VMEM((_SUB, lsz), jnp.int32),
        ],
    )(locs_f, scores_f, boxes, labels.reshape(bsz, 1, n_obj), priors_f)

    conf2 = conf_neg.reshape(bsz, p2)
    stats2 = stats.reshape(bsz, 128)

    out = pl.pallas_call(
        functools.partial(_stage2_body, p_real=p),
        out_shape=jax.ShapeDtypeStruct((1, 128), jnp.float32),
    )(conf2, stats2)

    total = out[0, 0]
    conf_loss = out[0, 1]
    loc_loss = out[0, 2]
    n_positives = stats2[:, 0].astype(jnp.int32)
    return total, conf_loss, loc_loss, n_positives


# speedup vs baseline: 1.0110x; 1.0110x over previous
import functools

import jax
import jax.numpy as jnp
from jax.experimental import pallas as pl
from jax.experimental.pallas import tpu as pltpu

_THRESHOLD = 0.5
_NEG_POS_RATIO = 3.0
_ALPHA = 1.0
_SUB = 8
_CH = 384


def _stage1_body(locs_ref, scores_ref, boxes_ref, labels_ref, priors_ref,
                 conf_ref, stats_ref, ov_s, ofp_s, obj_s, *, n_obj, p_real):
    s, l = priors_ref.shape[1], priors_ref.shape[2]
    p2 = s * l
    nch = l // _CH

    def box_scalars(j):
        bx0 = boxes_ref[0, j, 0]
        by0 = boxes_ref[0, j, 1]
        bx1 = boxes_ref[0, j, 2]
        by1 = boxes_ref[0, j, 3]
        return bx0, by0, bx1, by1

    for c in range(nch):
        cs = slice(c * _CH, (c + 1) * _CH)
        pcx = priors_ref[0, :, cs]
        pcy = priors_ref[1, :, cs]
        pw = priors_ref[2, :, cs]
        ph = priors_ref[3, :, cs]
        px0 = pcx - pw / 2.0
        py0 = pcy - ph / 2.0
        px1 = pcx + pw / 2.0
        py1 = pcy + ph / 2.0
        a2 = (px1 - px0) * (py1 - py0)
        vals = []
        idxs = []
        for j in range(n_obj):
            bx0, by0, bx1, by1 = box_scalars(j)
            iw = jnp.maximum(
                jnp.minimum(bx1, px1) - jnp.maximum(bx0, px0), 0.0)
            ih = jnp.maximum(
                jnp.minimum(by1, py1) - jnp.maximum(by0, py0), 0.0)
            inter = iw * ih
            a1 = (bx1 - bx0) * (by1 - by0)
            ov = inter / (a1 + a2 - inter)
            ov_s[j, :, cs] = ov
            vals.append(ov)
            idxs.append(j)
        while len(vals) > 1:
            nv, ni = [], []
            for t in range(0, len(vals), 2):
                keep = vals[t] >= vals[t + 1]
                nv.append(jnp.maximum(vals[t], vals[t + 1]))
                a, b = idxs[t], idxs[t + 1]
                if isinstance(a, int):
                    a = jnp.full((s, _CH), a, jnp.int32)
                ni.append(jnp.where(keep, a, b))
            vals, idxs = nv, ni
        ofp_s[:, cs] = vals[0]
        obj_s[:, cs] = idxs[0]

    p_idx = (jax.lax.broadcasted_iota(jnp.int32, (s, l), 0) * l +
             jax.lax.broadcasted_iota(jnp.int32, (s, l), 1))
    pfo = []
    for j in range(n_obj):
        ovj = ov_s[j]
        rmax = jnp.max(ovj)
        pfo.append(jnp.min(jnp.where(ovj >= rmax, p_idx, p2)))

    acc_npos = jnp.zeros((s, _CH), jnp.float32)
    acc_cpos = jnp.zeros((s, _CH), jnp.float32)
    acc_labs = jnp.zeros((s, _CH), jnp.float32)
    for c in range(nch):
        cs = slice(c * _CH, (c + 1) * _CH)
        p_idx_c = (jax.lax.broadcasted_iota(jnp.int32, (s, _CH), 0) * l +
                   jax.lax.broadcasted_iota(jnp.int32, (s, _CH), 1) + c * _CH)
        ofp_c = ofp_s[:, cs]
        obj_c = obj_s[:, cs]
        fj = [jnp.where(p_idx_c == pfo[j], j, -1) for j in range(n_obj)]
        while len(fj) > 1:
            fj = [jnp.maximum(fj[t], fj[t + 1]) for t in range(0, len(fj), 2)]
        fj = fj[0]
        forced = fj >= 0
        obj_c = jnp.where(forced, fj, obj_c)
        ofp_c = jnp.where(forced, 1.0, ofp_c)
        bbits = [((obj_c >> b) & 1) == 1 for b in range(4)]

        def _mux(table):
            lvl = list(table)
            for b in range(4):
                lvl = [jnp.where(bbits[b], lvl[t + 1], lvl[t])
                       for t in range(0, len(lvl), 2)]
            return lvl[0]

        lab = _mux([labels_ref[0, 0, j] for j in range(n_obj)])
        mx0 = _mux([boxes_ref[0, j, 0] for j in range(n_obj)])
        my0 = _mux([boxes_ref[0, j, 1] for j in range(n_obj)])
        mx1 = _mux([boxes_ref[0, j, 2] for j in range(n_obj)])
        my1 = _mux([boxes_ref[0, j, 3] for j in range(n_obj)])

        tc = jnp.where(ofp_c < _THRESHOLD, 0, lab)
        posm = (tc > 0) & (p_idx_c < p_real)
        posf = jnp.where(posm, 1.0, 0.0)
        acc_npos = acc_npos + posf

        pcx = priors_ref[0, :, cs]
        pcy = priors_ref[1, :, cs]
        pw = priors_ref[2, :, cs]
        ph = priors_ref[3, :, cs]
        cx = (mx0 + mx1) / 2.0
        cy = (my0 + my1) / 2.0
        w = mx1 - mx0
        h = my1 - my0
        g0 = (cx - pcx) / (pw / 10.0)
        g1 = (cy - pcy) / (ph / 10.0)
        g2 = jnp.log(w / pw) * 5.0
        g3 = jnp.log(h / ph) * 5.0
        loc_abs = (jnp.abs(locs_ref[0, 0, :, cs] - g0) +
                   jnp.abs(locs_ref[0, 1, :, cs] - g1) +
                   jnp.abs(locs_ref[0, 2, :, cs] - g2) +
                   jnp.abs(locs_ref[0, 3, :, cs] - g3))
        acc_labs = acc_labs + jnp.where(posm, loc_abs, 0.0)

        l0 = scores_ref[0, 0, :, cs]
        l1 = scores_ref[0, 1, :, cs]
        l2 = scores_ref[0, 2, :, cs]

        def _sp(x):
            return jnp.maximum(x, 0.0) + jnp.log1p(jnp.exp(-jnp.abs(x)))

        s_all = _sp(l0) + _sp(l1) + _sp(l2)
        d = jnp.where(tc == 0, l0,
                      jnp.where(tc == 1, l1,
                                jnp.where(tc == 2, l2, l1 + l2)))
        conf_all = s_all - d
        acc_cpos = acc_cpos + jnp.where(posm, conf_all, 0.0)
        conf_ref[0, :, cs] = jnp.where(
            posm | (p_idx_c >= p_real), 0.0, conf_all)

    n_pos = jnp.sum(acc_npos)
    conf_pos_row = jnp.sum(acc_cpos)
    loc_abs_row = jnp.sum(acc_labs)
    lane = jax.lax.broadcasted_iota(jnp.int32, (1, 128), 1)
    stats_ref[0] = jnp.where(
        lane == 0, n_pos,
        jnp.where(lane == 1, conf_pos_row,
                  jnp.where(lane == 2, loc_abs_row, 0.0)))


def _stage2_body(conf_ref, stats_ref, out_ref, *, p_real):
    b = conf_ref.shape[0]
    conf = conf_ref[...]
    stats = stats_ref[...]
    n_pos = stats[:, 0:1]
    npt = jnp.sum(n_pos)
    kf = jnp.minimum(n_pos * _NEG_POS_RATIO, float(p_real))

    bits = jax.lax.bitcast_convert_type(conf, jnp.int32)
    lo0 = jnp.full((b, 1), -1, jnp.int32)
    hi0 = jnp.full((b, 1), jnp.iinfo(jnp.int32).max - 1, jnp.int32)

    def _it(_, carry):
        lo, hi = carry
        mid = lo + ((hi - lo) >> 1)
        cnt = jnp.sum(jnp.where(bits > mid, 1.0, 0.0),
                      axis=1, keepdims=True)
        take_hi = cnt >= kf
        return jnp.where(take_hi, mid, lo), jnp.where(take_hi, hi, mid)

    _, hi = jax.lax.fori_loop(0, 31, _it, (lo0, hi0))
    vk_bits = hi
    vk = jax.lax.bitcast_convert_type(vk_bits, jnp.float32)
    gt = bits > vk_bits
    cnt_g = jnp.sum(jnp.where(gt, 1.0, 0.0), axis=1, keepdims=True)
    sum_g = jnp.sum(jnp.where(gt, conf, 0.0), axis=1, keepdims=True)
    hard = jnp.where(kf > 0.0, sum_g + vk * (kf - cnt_g), 0.0)

    conf_pos = jnp.sum(stats[:, 1:2])
    loc_abs = jnp.sum(stats[:, 2:3])
    conf_loss = (jnp.sum(hard) + conf_pos) / (1e-10 + npt)
    loc_loss = jnp.where(npt > 0.0,
                         loc_abs / (4.0 * jnp.maximum(npt, 1.0)), 0.0)
    total = conf_loss + _ALPHA * loc_loss

    lane = jax.lax.broadcasted_iota(jnp.int32, (1, 128), 1)
    out_ref[...] = jnp.where(
        lane == 0, total,
        jnp.where(lane == 1, conf_loss,
                  jnp.where(lane == 2, loc_loss, 0.0)))


def kernel(predicted_locs, predicted_scores, boxes, labels, priors_cxcy):
    bsz, p, _ = predicted_scores.shape
    n_obj = boxes.shape[1]
    p2 = p + (-p % (_SUB * 128))
    lsz = p2 // _SUB

    def _fold(x_t):
        pads = [(0, 0)] * (x_t.ndim - 1) + [(0, p2 - p)]
        return jnp.pad(x_t, pads).reshape(x_t.shape[:-1] + (_SUB, lsz))

    locs_f = _fold(jnp.transpose(predicted_locs, (0, 2, 1)))
    scores_f = _fold(jnp.transpose(predicted_scores, (0, 2, 1)))
    priors_f = _fold(jnp.transpose(priors_cxcy, (1, 0)))

    conf_neg, stats = pl.pallas_call(
        functools.partial(_stage1_body, n_obj=n_obj, p_real=p),
        grid=(bsz,),
        in_specs=[
            pl.BlockSpec((1, 4, _SUB, lsz), lambda i: (i, 0, 0, 0)),
            pl.BlockSpec((1, 3, _SUB, lsz), lambda i: (i, 0, 0, 0)),
            pl.BlockSpec((1, n_obj, 4), lambda i: (i, 0, 0),
                         memory_space=pltpu.SMEM),
            pl.BlockSpec((1, 1, n_obj), lambda i: (i, 0, 0),
                         memory_space=pltpu.SMEM),
            pl.BlockSpec((4, _SUB, lsz), lambda i: (0, 0, 0)),
        ],
        out_specs=[
            pl.BlockSpec((1, _SUB, lsz), lambda i: (i, 0, 0)),
            pl.BlockSpec((1, 1, 128), lambda i: (i, 0, 0)),
        ],
        out_shape=[
            jax.ShapeDtypeStruct((bsz, _SUB, lsz), jnp.float32),
            jax.ShapeDtypeStruct((bsz, 1, 128), jnp.float32),
        ],
        scratch_shapes=[
            pltpu.VMEM((n_obj, _SUB, lsz), jnp.float32),
            pltpu.VMEM((_SUB, lsz), jnp.float32),
            pltpu.VMEM((_SUB, lsz), jnp.int32),
        ],
    )(locs_f, scores_f, boxes, labels.reshape(bsz, 1, n_obj), priors_f)

    conf2 = conf_neg.reshape(bsz, p2)
    stats2 = stats.reshape(bsz, 128)

    out = pl.pallas_call(
        functools.partial(_stage2_body, p_real=p),
        out_shape=jax.ShapeDtypeStruct((1, 128), jnp.float32),
    )(conf2, stats2)

    total = out[0, 0]
    conf_loss = out[0, 1]
    loc_loss = out[0, 2]
    n_positives = stats2[:, 0].astype(jnp.int32)
    return total, conf_loss, loc_loss, n_positives
